# baseline (device time: 42629 ns/iter reference)
import jax
import jax.numpy as jnp
from jax import lax
from jax.experimental import pallas as pl
from jax.experimental.pallas import tpu as pltpu

N_DEV = 32
N_SUB = 16
GRP = 4
N_GRP = N_SUB // GRP

OFFS = [8, 7, 9, 6, 10, 5, 11, 4, 12, 3, 13, 2, 14, 1, 15]
OFFS_ALL = OFFS + [N_SUB]


def kernel(x, w_mat):
    m, n = x.shape[0], w_mat.shape[1]
    m_per = m // N_DEV

    def g(r, x_coord):
        return (r // 4) * 8 + 2 * (r % 4) + ((x_coord + (r % 4)) % 2)

    def body(
        x_ref, w_ref, out_ref,
        p1_stage, p1_buf, stage_buf, p2_buf,
        p1_ssems, p1_rsems, p2_ssems, p2_rsems,
    ):
        me = lax.axis_index("i")
        s_me = me % 8
        z_me = me // 8
        y_me = s_me // 2
        x_me = ((s_me + 1) // 2) % 2
        r_me = z_me * 4 + y_me
        nbr = z_me * 8 + 2 * y_me + ((x_me + 1 + y_me) % 2)

        barrier_sem = pltpu.get_barrier_semaphore()
        pl.semaphore_signal(
            barrier_sem, inc=1,
            device_id=(nbr,), device_id_type=pl.DeviceIdType.MESH,
        )
        for o in OFFS:
            peer = g((r_me + o) % N_SUB, x_me)
            pl.semaphore_signal(
                barrier_sem, inc=1,
                device_id=(peer,), device_id_type=pl.DeviceIdType.MESH,
            )
        pl.semaphore_wait(barrier_sem, N_SUB)

        p1_rdmas = []
        for grp in range(N_GRP):
            for j in range(GRP):
                k = grp * GRP + j
                sig = (r_me + OFFS_ALL[k]) % N_SUB
                c_nbr = g(sig, 1 - x_me)
                p1_stage[k] = jnp.dot(
                    x_ref[pl.ds(c_nbr * m_per, m_per), :], w_ref[...],
                    preferred_element_type=jnp.float32,
                )
            rdma = pltpu.make_async_remote_copy(
                src_ref=p1_stage.at[pl.ds(grp * GRP, GRP)],
                dst_ref=p1_buf.at[pl.ds(grp * GRP, GRP)],
                send_sem=p1_ssems.at[grp],
                recv_sem=p1_rsems.at[grp],
                device_id=(nbr,),
                device_id_type=pl.DeviceIdType.MESH,
            )
            rdma.start()
            p1_rdmas.append(rdma)

        p2_rdmas = []
        for k, o in enumerate(OFFS):
            if k % GRP == 0:
                p1_rdmas[k // GRP].wait_recv()
            sig = (r_me + o) % N_SUB
            t = g(sig, x_me)
            stage_buf[k] = (
                jnp.dot(
                    x_ref[pl.ds(t * m_per, m_per), :], w_ref[...],
                    preferred_element_type=jnp.float32,
                )
                + p1_buf[k]
            )
            rdma = pltpu.make_async_remote_copy(
                src_ref=stage_buf.at[k],
                dst_ref=p2_buf.at[r_me],
                send_sem=p2_ssems.at[k],
                recv_sem=p2_rsems.at[r_me],
                device_id=(t,),
                device_id_type=pl.DeviceIdType.MESH,
            )
            rdma.start()
            p2_rdmas.append(rdma)

        out_ref[...] = (
            jnp.dot(
                x_ref[pl.ds(me * m_per, m_per), :], w_ref[...],
                preferred_element_type=jnp.float32,
            )
            + p1_buf[N_SUB - 1]
        )

        for o in OFFS:
            rho = (r_me - o) % N_SUB
            recv = pltpu.make_async_remote_copy(
                src_ref=p2_buf.at[rho],
                dst_ref=p2_buf.at[rho],
                send_sem=p2_rsems.at[rho],
                recv_sem=p2_rsems.at[rho],
                device_id=(me,),
                device_id_type=pl.DeviceIdType.MESH,
            )
            recv.wait_recv()
            out_ref[...] += p2_buf[pl.ds(rho, 1), :, :][0]

        for rdma in p1_rdmas:
            rdma.wait_send()
        for rdma in p2_rdmas:
            rdma.wait_send()

    return pl.pallas_call(
        body,
        out_shape=jax.ShapeDtypeStruct((m_per, n), jnp.float32),
        in_specs=[
            pl.BlockSpec(memory_space=pltpu.VMEM),
            pl.BlockSpec(memory_space=pltpu.VMEM),
        ],
        out_specs=pl.BlockSpec(memory_space=pltpu.VMEM),
        scratch_shapes=[
            pltpu.VMEM((N_SUB, m_per, n), jnp.float32),
            pltpu.VMEM((N_SUB, m_per, n), jnp.float32),
            pltpu.VMEM((N_SUB - 1, m_per, n), jnp.float32),
            pltpu.VMEM((N_SUB, m_per, n), jnp.float32),
            pltpu.SemaphoreType.DMA((N_GRP,)),
            pltpu.SemaphoreType.DMA((N_GRP,)),
            pltpu.SemaphoreType.DMA((N_SUB - 1,)),
            pltpu.SemaphoreType.DMA((N_SUB,)),
        ],
        compiler_params=pltpu.CompilerParams(collective_id=0),
    )(x, w_mat)


# device time: 40385 ns/iter; 1.0556x vs baseline; 1.0556x over previous
import jax
import jax.numpy as jnp
from jax import lax
from jax.experimental import pallas as pl
from jax.experimental.pallas import tpu as pltpu

N_DEV = 32
N_SUB = 16
GRP = 1
N_GRP = N_SUB // GRP

OFFS = [8, 7, 9, 6, 10, 5, 11, 4, 12, 3, 13, 2, 14, 1, 15]
OFFS_ALL = OFFS + [N_SUB]


def kernel(x, w_mat):
    m, n = x.shape[0], w_mat.shape[1]
    m_per = m // N_DEV

    def g(r, x_coord):
        return (r // 4) * 8 + 2 * (r % 4) + ((x_coord + (r % 4)) % 2)

    def body(
        x_ref, w_ref, out_ref,
        p1_stage, p1_buf, stage_buf, p2_buf,
        p1_ssems, p1_rsems, p2_ssems, p2_rsems,
    ):
        me = lax.axis_index("i")
        s_me = me % 8
        z_me = me // 8
        y_me = s_me // 2
        x_me = ((s_me + 1) // 2) % 2
        r_me = z_me * 4 + y_me
        nbr = z_me * 8 + 2 * y_me + ((x_me + 1 + y_me) % 2)

        barrier_sem = pltpu.get_barrier_semaphore()
        pl.semaphore_signal(
            barrier_sem, inc=1,
            device_id=(nbr,), device_id_type=pl.DeviceIdType.MESH,
        )
        for o in OFFS:
            peer = g((r_me + o) % N_SUB, x_me)
            pl.semaphore_signal(
                barrier_sem, inc=1,
                device_id=(peer,), device_id_type=pl.DeviceIdType.MESH,
            )
        pl.semaphore_wait(barrier_sem, N_SUB)

        p1_rdmas = []
        for grp in range(N_GRP):
            for j in range(GRP):
                k = grp * GRP + j
                sig = (r_me + OFFS_ALL[k]) % N_SUB
                c_nbr = g(sig, 1 - x_me)
                p1_stage[k] = jnp.dot(
                    x_ref[pl.ds(c_nbr * m_per, m_per), :], w_ref[...],
                    preferred_element_type=jnp.float32,
                )
            rdma = pltpu.make_async_remote_copy(
                src_ref=p1_stage.at[pl.ds(grp * GRP, GRP)],
                dst_ref=p1_buf.at[pl.ds(grp * GRP, GRP)],
                send_sem=p1_ssems.at[grp],
                recv_sem=p1_rsems.at[grp],
                device_id=(nbr,),
                device_id_type=pl.DeviceIdType.MESH,
            )
            rdma.start()
            p1_rdmas.append(rdma)

        p2_rdmas = []
        for k, o in enumerate(OFFS):
            if k % GRP == 0:
                p1_rdmas[k // GRP].wait_recv()
            sig = (r_me + o) % N_SUB
            t = g(sig, x_me)
            stage_buf[k] = (
                jnp.dot(
                    x_ref[pl.ds(t * m_per, m_per), :], w_ref[...],
                    preferred_element_type=jnp.float32,
                )
                + p1_buf[k]
            )
            rdma = pltpu.make_async_remote_copy(
                src_ref=stage_buf.at[k],
                dst_ref=p2_buf.at[r_me],
                send_sem=p2_ssems.at[k],
                recv_sem=p2_rsems.at[r_me],
                device_id=(t,),
                device_id_type=pl.DeviceIdType.MESH,
            )
            rdma.start()
            p2_rdmas.append(rdma)

        for grp in range((N_SUB - 1 + GRP - 1) // GRP, N_GRP):
            p1_rdmas[grp].wait_recv()
        out_ref[...] = (
            jnp.dot(
                x_ref[pl.ds(me * m_per, m_per), :], w_ref[...],
                preferred_element_type=jnp.float32,
            )
            + p1_buf[N_SUB - 1]
        )

        for o in OFFS:
            rho = (r_me - o) % N_SUB
            recv = pltpu.make_async_remote_copy(
                src_ref=p2_buf.at[rho],
                dst_ref=p2_buf.at[rho],
                send_sem=p2_rsems.at[rho],
                recv_sem=p2_rsems.at[rho],
                device_id=(me,),
                device_id_type=pl.DeviceIdType.MESH,
            )
            recv.wait_recv()
            out_ref[...] += p2_buf[pl.ds(rho, 1), :, :][0]

        for rdma in p1_rdmas:
            rdma.wait_send()
        for rdma in p2_rdmas:
            rdma.wait_send()

    return pl.pallas_call(
        body,
        out_shape=jax.ShapeDtypeStruct((m_per, n), jnp.float32),
        in_specs=[
            pl.BlockSpec(memory_space=pltpu.VMEM),
            pl.BlockSpec(memory_space=pltpu.VMEM),
        ],
        out_specs=pl.BlockSpec(memory_space=pltpu.VMEM),
        scratch_shapes=[
            pltpu.VMEM((N_SUB, m_per, n), jnp.float32),
            pltpu.VMEM((N_SUB, m_per, n), jnp.float32),
            pltpu.VMEM((N_SUB - 1, m_per, n), jnp.float32),
            pltpu.VMEM((N_SUB, m_per, n), jnp.float32),
            pltpu.SemaphoreType.DMA((N_GRP,)),
            pltpu.SemaphoreType.DMA((N_GRP,)),
            pltpu.SemaphoreType.DMA((N_SUB - 1,)),
            pltpu.SemaphoreType.DMA((N_SUB,)),
        ],
        compiler_params=pltpu.CompilerParams(collective_id=0),
    )(x, w_mat)
